# baseline (speedup 1.0000x reference)
import jax
import jax.numpy as jnp
import numpy as np
from jax.experimental import pallas as pl

_ANCHORS = np.array(
    [[12, 16], [19, 36], [40, 28], [36, 75], [76, 55], [72, 146],
     [142, 110], [192, 243], [459, 401]], dtype=np.float32)
_ANCHOR_MASKS = [[6, 7, 8], [3, 4, 5], [0, 1, 2]]
_DOWNSAMPLE = [32.0, 16.0, 8.0]
_OC = 85
_CHUNK = 256


def _decode_chunk(y, a, row0, ds, anc):
    _, r_dim, c_dim = y.shape
    c = jax.lax.broadcasted_iota(jnp.int32, y.shape, 0)
    gy = (row0 + jax.lax.broadcasted_iota(jnp.int32, y.shape, 1)
          ).astype(jnp.float32)
    gx = jax.lax.broadcasted_iota(jnp.int32, y.shape, 2).astype(jnp.float32)
    sig = jax.nn.sigmoid(y)
    ex = jnp.exp(y)
    g = jnp.where(c == 0, gx, gy)
    aw = jnp.where(a == 0, anc[0][0], jnp.where(a == 1, anc[1][0], anc[2][0]))
    ah = jnp.where(a == 0, anc[0][1], jnp.where(a == 1, anc[1][1], anc[2][1]))
    av = jnp.where(c == 2, aw, ah)
    xywh = jnp.where(c < 2, (sig + g) * ds, ex * av)
    out = jnp.where(c < 4, xywh, sig)
    return jnp.transpose(out, (1, 2, 0)).reshape(r_dim * c_dim, _OC)


def _body(x0_ref, x1_ref, x2_ref, o_ref):
    j = pl.program_id(1)

    @pl.when(j < 3)
    def _():
        o_ref[0] = _decode_chunk(x0_ref[0], j, 0, _DOWNSAMPLE[0],
                                 _ANCHORS[np.array(_ANCHOR_MASKS[0])])

    @pl.when((j >= 3) & (j < 15))
    def _():
        jj = j - 3
        o_ref[0] = _decode_chunk(x1_ref[0], jj // 4, (jj % 4) * 8,
                                 _DOWNSAMPLE[1],
                                 _ANCHORS[np.array(_ANCHOR_MASKS[1])])

    @pl.when(j >= 15)
    def _():
        jj = j - 15
        s2 = jj % 16
        rows = _decode_chunk(x2_ref[0], jj // 16, (s2 // 2) * 8,
                             _DOWNSAMPLE[2],
                             _ANCHORS[np.array(_ANCHOR_MASKS[2])])

        @pl.when(s2 % 2 == 0)
        def _():
            o_ref[0] = rows[:_CHUNK]

        @pl.when(s2 % 2 == 1)
        def _():
            o_ref[0] = rows[_CHUNK:]


def _split2(j):
    jj = jnp.clip(j - 15, 0, 47)
    return (jj // 16, (jj % 16) // 2)


def kernel(x0, x1, x2):
    bs = x0.shape[0]
    n_chunks = 63
    out = pl.pallas_call(
        _body,
        grid=(bs, n_chunks),
        in_specs=[
            pl.BlockSpec((1, _OC, 16, 16),
                         lambda b, j: (b, jnp.minimum(j, 2), 0, 0)),
            pl.BlockSpec((1, _OC, 8, 32),
                         lambda b, j: (b,) + _split1(j) + (0,)),
            pl.BlockSpec((1, _OC, 8, 64),
                         lambda b, j: (b,) + _split2(j) + (0,)),
        ],
        out_specs=pl.BlockSpec((1, _CHUNK, _OC), lambda b, j: (b, j, 0)),
        out_shape=jax.ShapeDtypeStruct((bs, n_chunks * _CHUNK, _OC),
                                       jnp.float32),
    )(x0, x1, x2)
    return out


def _split1(j):
    jj = jnp.clip(j - 3, 0, 11)
    return (jj // 4, jj % 4)


# speedup vs baseline: 2.1178x; 2.1178x over previous
import functools

import jax
import jax.numpy as jnp
import numpy as np
from jax.experimental import pallas as pl

_ANCHORS = np.array(
    [[12, 16], [19, 36], [40, 28], [36, 75], [76, 55], [72, 146],
     [142, 110], [192, 243], [459, 401]], dtype=np.float32)
_ANCHOR_MASKS = [[6, 7, 8], [3, 4, 5], [0, 1, 2]]
_DOWNSAMPLE = [32.0, 16.0, 8.0]
_OC = 85
_CHUNK = 256


def _decode_chunk(y, a, p0, nx, ds, anc):
    c = jax.lax.broadcasted_iota(jnp.int32, y.shape, 0)
    p = p0 + jax.lax.broadcasted_iota(jnp.int32, y.shape, 1)
    gx = (p % nx).astype(jnp.float32)
    gy = (p // nx).astype(jnp.float32)
    sig = jax.nn.sigmoid(y)
    ex = jnp.exp(y)
    g = jnp.where(c == 0, gx, gy)
    aw = jnp.where(a == 0, anc[0][0], jnp.where(a == 1, anc[1][0], anc[2][0]))
    ah = jnp.where(a == 0, anc[0][1], jnp.where(a == 1, anc[1][1], anc[2][1]))
    av = jnp.where(c == 2, aw, ah)
    xywh = jnp.where(c < 2, (sig + g) * ds, ex * av)
    out = jnp.where(c < 4, xywh, sig)
    return out.T


def _body(x0_ref, x1_ref, x2_ref, o_ref):
    i = pl.program_id(0)
    k = pl.program_id(1)
    a = jax.lax.rem(i, 3)

    @pl.when(k == 0)
    def _():
        o_ref[0] = _decode_chunk(
            x0_ref[0], a, 0, 16, _DOWNSAMPLE[0],
            _ANCHORS[np.array(_ANCHOR_MASKS[0])])

    @pl.when((k >= 1) & (k < 5))
    def _():
        o_ref[0] = _decode_chunk(
            x1_ref[0], a, (k - 1) * _CHUNK, 32, _DOWNSAMPLE[1],
            _ANCHORS[np.array(_ANCHOR_MASKS[1])])

    @pl.when(k >= 5)
    def _():
        o_ref[0] = _decode_chunk(
            x2_ref[0], a, (k - 5) * _CHUNK, 64, _DOWNSAMPLE[2],
            _ANCHORS[np.array(_ANCHOR_MASKS[2])])


def _out_chunk(i, k):
    b = i // 3
    a = i % 3
    s1 = jnp.clip(k - 1, 0, 3)
    s2 = jnp.clip(k - 5, 0, 15)
    off = jnp.where(k == 0, a,
                    jnp.where(k < 5, 3 + a * 4 + s1, 15 + a * 16 + s2))
    return b * 63 + off


def kernel(x0, x1, x2):
    bs = x0.shape[0]
    xr = [x.reshape(bs * 3, _OC, x.shape[-1] * x.shape[-2])
          for x in (x0, x1, x2)]
    out = pl.pallas_call(
        _body,
        grid=(bs * 3, 21),
        in_specs=[
            pl.BlockSpec((1, _OC, 256), lambda i, k: (i, 0, 0)),
            pl.BlockSpec((1, _OC, 256),
                         lambda i, k: (i, 0, jnp.clip(k - 1, 0, 3))),
            pl.BlockSpec((1, _OC, 256),
                         lambda i, k: (i, 0, jnp.clip(k - 5, 0, 15))),
        ],
        out_specs=pl.BlockSpec((1, _CHUNK, _OC),
                               lambda i, k: (_out_chunk(i, k), 0, 0)),
        out_shape=jax.ShapeDtypeStruct((bs * 63, _CHUNK, _OC), jnp.float32),
    )(*xr)
    return out.reshape(bs, 63 * _CHUNK, _OC)


# speedup vs baseline: 5.0129x; 2.3671x over previous
import jax
import jax.numpy as jnp
import numpy as np
from jax.experimental import pallas as pl
from jax.experimental.pallas import tpu as pltpu

_ANCHORS = np.array(
    [[12, 16], [19, 36], [40, 28], [36, 75], [76, 55], [72, 146],
     [142, 110], [192, 243], [459, 401]], dtype=np.float32)
_ANCHOR_MASKS = [[6, 7, 8], [3, 4, 5], [0, 1, 2]]
_DOWNSAMPLE = [32.0, 16.0, 8.0]
_OC = 85
_ROW0 = (0, 768, 3840)


def _decode(y, a, nx, ds, anc):
    c = jax.lax.broadcasted_iota(jnp.int32, y.shape, 0)
    p = jax.lax.broadcasted_iota(jnp.int32, y.shape, 1)
    gx = (p % nx).astype(jnp.float32)
    gy = (p // nx).astype(jnp.float32)
    sig = jax.nn.sigmoid(y)
    ex = jnp.exp(y)
    g = jnp.where(c == 0, gx, gy)
    aw = jnp.where(a == 0, anc[0][0], jnp.where(a == 1, anc[1][0], anc[2][0]))
    ah = jnp.where(a == 0, anc[0][1], jnp.where(a == 1, anc[1][1], anc[2][1]))
    av = jnp.where(c == 2, aw, ah)
    xywh = jnp.where(c < 2, (sig + g) * ds, ex * av)
    return jnp.where(c < 4, xywh, sig).T


def _body(x0_ref, x1_ref, x2_ref, o_ref, s0, s1, s2, m0, m1, m2):
    i = pl.program_id(0)
    b = i // 3
    a = jax.lax.rem(i, 3)
    copies = []
    for sc, (x_ref, s_ref, m_ref) in enumerate(
            ((x0_ref, s0, m0), (x1_ref, s1, m1), (x2_ref, s2, m2))):
        ny, nx = x_ref.shape[2], x_ref.shape[3]
        npix = ny * nx
        y = x_ref[0].reshape(_OC, npix)
        s_ref[...] = _decode(y, a, nx, _DOWNSAMPLE[sc],
                             _ANCHORS[np.array(_ANCHOR_MASKS[sc])])
        cp = pltpu.make_async_copy(
            s_ref, o_ref.at[b, pl.ds(_ROW0[sc] + a * npix, npix), :], m_ref)
        cp.start()
        copies.append(cp)
    for cp in copies:
        cp.wait()


def kernel(x0, x1, x2):
    bs = x0.shape[0]
    views = [x.reshape(bs * 3, _OC, x.shape[-2], x.shape[-1])
             for x in (x0, x1, x2)]
    out = pl.pallas_call(
        _body,
        grid=(bs * 3,),
        in_specs=[
            pl.BlockSpec((1, _OC, 16, 16), lambda i: (i, 0, 0, 0)),
            pl.BlockSpec((1, _OC, 32, 32), lambda i: (i, 0, 0, 0)),
            pl.BlockSpec((1, _OC, 64, 64), lambda i: (i, 0, 0, 0)),
        ],
        out_specs=pl.BlockSpec(memory_space=pl.ANY),
        out_shape=jax.ShapeDtypeStruct((bs, 16128, _OC), jnp.float32),
        scratch_shapes=[
            pltpu.VMEM((256, _OC), jnp.float32),
            pltpu.VMEM((1024, _OC), jnp.float32),
            pltpu.VMEM((4096, _OC), jnp.float32),
            pltpu.SemaphoreType.DMA,
            pltpu.SemaphoreType.DMA,
            pltpu.SemaphoreType.DMA,
        ],
    )(*views)
    return out


# speedup vs baseline: 15.6785x; 3.1276x over previous
import jax
import jax.numpy as jnp
import numpy as np
from jax.experimental import pallas as pl

_ANCHORS = np.array(
    [[12, 16], [19, 36], [40, 28], [36, 75], [76, 55], [72, 146],
     [142, 110], [192, 243], [459, 401]], dtype=np.float32)
_ANCHOR_MASKS = [[6, 7, 8], [3, 4, 5], [0, 1, 2]]
_DOWNSAMPLE = [32.0, 16.0, 8.0]
_OC = 85
_NC = 255


def _emit(x_ref, o_ref, a, s, nx, ds, anc):
    y = x_ref[...]
    ci = jax.lax.broadcasted_iota(jnp.int32, (_OC, _NC), 0)
    ki = jax.lax.broadcasted_iota(jnp.int32, (_OC, _NC), 1)
    e = (ki == ci + a * _OC).astype(jnp.float32)
    raw = jax.lax.dot_general(e, y, (((1,), (2,)), ((), ())),
                              precision=jax.lax.Precision.DEFAULT)
    sig = jax.nn.sigmoid(raw)
    head = raw[0:8]
    c = jax.lax.broadcasted_iota(jnp.int32, head.shape, 0)
    p = s * 256 + jax.lax.broadcasted_iota(jnp.int32, head.shape, 2)
    gx = (p % nx).astype(jnp.float32)
    gy = (p // nx).astype(jnp.float32)
    g = jnp.where(c == 0, gx, gy)
    aw = jnp.where(a == 0, anc[0][0], jnp.where(a == 1, anc[1][0], anc[2][0]))
    ah = jnp.where(a == 0, anc[0][1], jnp.where(a == 1, anc[1][1], anc[2][1]))
    av = jnp.where(c == 2, aw, ah)
    xywh = jnp.where(c < 2, (sig[0:8] + g) * ds, jnp.exp(head) * av)
    o_ref[0:8] = jnp.where(c < 4, xywh, sig[0:8])
    o_ref[8:_OC] = sig[8:_OC]


def _body(x0_ref, x1_ref, x2_ref, o_ref):
    j = pl.program_id(0)

    @pl.when(j < 3)
    def _():
        _emit(x0_ref, o_ref, j, 0, 16, _DOWNSAMPLE[0],
              _ANCHORS[np.array(_ANCHOR_MASKS[0])])

    @pl.when((j >= 3) & (j < 15))
    def _():
        t = j - 3
        _emit(x1_ref, o_ref, jax.lax.rem(t, 3), t // 3, 32, _DOWNSAMPLE[1],
              _ANCHORS[np.array(_ANCHOR_MASKS[1])])

    @pl.when(j >= 15)
    def _():
        t = j - 15
        _emit(x2_ref, o_ref, jax.lax.rem(t, 3), t // 3, 64, _DOWNSAMPLE[2],
              _ANCHORS[np.array(_ANCHOR_MASKS[2])])


def _out_chunk(j):
    t1 = jnp.clip(j - 3, 0, 11)
    t2 = jnp.clip(j - 15, 0, 47)
    return jnp.where(
        j < 3, j,
        jnp.where(j < 15,
                  3 + (t1 % 3) * 4 + t1 // 3,
                  15 + (t2 % 3) * 16 + t2 // 3))


def kernel(x0, x1, x2):
    bs = x0.shape[0]
    xt = [
        jnp.transpose(x, (0, 2, 3, 1)).reshape(
            bs, x.shape[-2] * x.shape[-1], _NC)
        for x in (x0, x1, x2)
    ]
    o = pl.pallas_call(
        _body,
        grid=(63,),
        in_specs=[
            pl.BlockSpec((bs, 256, _NC), lambda j: (0, 0, 0)),
            pl.BlockSpec((bs, 256, _NC),
                         lambda j: (0, jnp.clip(j - 3, 0, 11) // 3, 0)),
            pl.BlockSpec((bs, 256, _NC),
                         lambda j: (0, jnp.clip(j - 15, 0, 47) // 3, 0)),
        ],
        out_specs=pl.BlockSpec((_OC, bs, 256), lambda j: (0, 0, _out_chunk(j))),
        out_shape=jax.ShapeDtypeStruct((_OC, bs, 63 * 256), jnp.float32),
    )(*xt)
    return jnp.transpose(o, (1, 2, 0))
